# baseline (speedup 1.0000x reference)
import functools

import jax
import jax.numpy as jnp
import numpy as np
from jax import lax
from jax.experimental import pallas as pl
from jax.experimental.pallas import tpu as pltpu
from jax.experimental.pallas import tpu_sc as plsc

N = 10000
E = 320000
D = 128

NC = 2
NS = 16
NW = NC * NS
EPT = E // NW
K = 40
NB = EPT // K
RPT = 624
REM = N - NS * RPT
IDXB = 14

_PERM = np.zeros((D,), np.int32)
for _q in range(D):
    _cc, _k = divmod(_q % 64, 16)
    _PERM[_q] = _cc * 32 + 2 * _k + (1 if _q >= 64 else 0)
_PMAT = np.zeros((D, D), np.float32)
_PMAT[np.arange(D), _PERM] = 1.0


_BN = 1000


def _matmul_body(x_ref, w_ref, o_ref):
    o_ref[...] = jnp.dot(x_ref[...], w_ref[...],
                         preferred_element_type=jnp.float32
                         ).astype(jnp.bfloat16)


def _matmul(x, W):
    return pl.pallas_call(
        _matmul_body,
        grid=(N // _BN,),
        in_specs=[
            pl.BlockSpec((_BN, D), lambda i: (i, 0)),
            pl.BlockSpec((D, D), lambda i: (0, 0)),
        ],
        out_specs=pl.BlockSpec((_BN, D), lambda i: (i, 0)),
        out_shape=jax.ShapeDtypeStruct((N, D), jnp.bfloat16),
    )(x, W)


def _scale_rows(rows_bf, scaled, w_all, bi):
    for g in range((K + 15) // 16):
        p = min(16, K - g * 16)
        lane0 = 16 - p
        wv = w_all[pl.ds(bi * K + g * 16 - lane0, 16)]
        for t in range(p):
            wj = wv[lane0 + t]
            j = g * 16 + t
            for cc in range(D // 32):
                v = rows_bf[j, pl.ds(cc * 16, 16)]
                a = lax.bitcast_convert_type(v << 16, jnp.float32)
                b = lax.bitcast_convert_type(v & jnp.int32(-65536),
                                             jnp.float32)
                scaled[j, pl.ds(cc * 16, 16)] = a * wj
                scaled[j, pl.ds(D // 2 + cc * 16, 16)] = b * wj


def _sc_body(support_hbm, sd_hbm, w_hbm, zeros_hbm, out_hbm,
             acc, sd_all, w_all, rb0, rb1, rb2, sc0, sc1, sc2,
             sblk, dblk, psem, g0, g1, g2, t0, t1, t2):
    c = lax.axis_index("c")
    s = lax.axis_index("s")
    wid = c * NS + s
    rbs = (rb0, rb1, rb2)
    scs = (sc0, sc1, sc2)
    gs = (g0, g1, g2)
    ts = (t0, t1, t2)

    d1 = pltpu.async_copy(sd_hbm.at[wid], sd_all, psem)
    d2 = pltpu.async_copy(w_hbm.at[wid], w_all, psem)
    d3 = pltpu.async_copy(zeros_hbm, acc.at[pl.ds(s * RPT, RPT)], psem)
    for u in range(3):
        pltpu.async_copy(zeros_hbm.at[pl.ds(0, K)], scs[u], ts[u])

    @pl.when(s == 0)
    def _():
        pltpu.sync_copy(zeros_hbm.at[pl.ds(0, REM)],
                        acc.at[pl.ds(NS * RPT, REM)])

    def unpack_src(bi, u):
        for off in (0, 16, K - 16):
            v = sd_all[bi, pl.ds(off, 16)]
            sblk[u, pl.ds(off, 16)] = v & ((1 << IDXB) - 1)

    def unpack_dst(bi, u):
        for off in (0, 16, K - 16):
            v = sd_all[bi, pl.ds(off, 16)]
            dblk[u, pl.ds(off, 16)] = v >> IDXB

    def gather(bi, u):
        pltpu.async_copy(support_hbm.at[sblk.at[u]], rbs[u], gs[u])

    def drain_gather(u):
        pltpu.make_async_copy(support_hbm.at[pl.ds(0, K)], rbs[u],
                              gs[u]).wait()

    def scatter(bi, u):
        pltpu.async_copy(scs[u], acc.at[dblk.at[u]], ts[u], add=True)

    def drain_scatter(u):
        pltpu.make_async_copy(zeros_hbm.at[pl.ds(0, K)], scs[u],
                              ts[u]).wait()

    d1.wait()
    d2.wait()
    d3.wait()
    unpack_src(0, 0)
    unpack_src(1, 1)
    gather(0, 0)
    gather(1, 1)
    plsc.subcore_barrier()

    def block_body(m, u):
        uz = (u + 2) % 3
        drain_gather(u)
        drain_scatter(u)
        _scale_rows(rbs[u], scs[u], w_all, m)

        @pl.when(m + 2 < NB)
        def _():
            unpack_src(m + 2, uz)
            gather(m + 2, uz)

        unpack_dst(m, u)
        scatter(m, u)

    block_body(0, 0)

    def triple(t, _):
        for i in range(3):
            block_body(3 * t + i + 1, (i + 1) % 3)
        return 0

    lax.fori_loop(0, (NB - 1) // 3, triple, 0)
    for u in range(3):
        drain_scatter(u)
    plsc.subcore_barrier()

    pltpu.sync_copy(acc.at[pl.ds(s * RPT, RPT)],
                    out_hbm.at[c, pl.ds(s * RPT, RPT)])

    @pl.when(s == 0)
    def _():
        pltpu.sync_copy(acc.at[pl.ds(NS * RPT, REM)],
                        out_hbm.at[c, pl.ds(NS * RPT, REM)])


def _sc_aggregate(support_bf, src, dst, w):
    mesh = plsc.VectorSubcoreMesh(core_axis_name="c", subcore_axis_name="s",
                                  num_cores=NC, num_subcores=NS)
    sd = src | (dst << IDXB)
    call = pl.kernel(
        _sc_body,
        out_type=jax.ShapeDtypeStruct((NC, N, D), jnp.float32),
        mesh=mesh,
        compiler_params=pltpu.CompilerParams(use_tc_tiling_on_sc=False),
        scratch_types=[
            pltpu.VMEM_SHARED((N, D), jnp.float32),
            pltpu.VMEM((NB, K), jnp.int32),
            pltpu.VMEM((EPT,), jnp.float32),
            pltpu.VMEM((K, D // 2), jnp.int32),
            pltpu.VMEM((K, D // 2), jnp.int32),
            pltpu.VMEM((K, D // 2), jnp.int32),
            pltpu.VMEM((K, D), jnp.float32),
            pltpu.VMEM((K, D), jnp.float32),
            pltpu.VMEM((K, D), jnp.float32),
            pltpu.VMEM((3, K), jnp.int32),
            pltpu.VMEM((3, K), jnp.int32),
            pltpu.SemaphoreType.DMA,
            pltpu.SemaphoreType.DMA,
            pltpu.SemaphoreType.DMA,
            pltpu.SemaphoreType.DMA,
            pltpu.SemaphoreType.DMA,
            pltpu.SemaphoreType.DMA,
            pltpu.SemaphoreType.DMA,
        ],
    )
    support_i32 = lax.bitcast_convert_type(
        support_bf.reshape(N, D // 2, 2), jnp.int32)
    return call(support_i32,
                sd.reshape(NW, NB, K),
                w.reshape(NW, EPT),
                jnp.zeros((RPT, D), jnp.float32))


def _combine_body(p_ref, pm_ref, b_ref, o_ref):
    o_ref[...] = jnp.dot(p_ref[0] + p_ref[1], pm_ref[...],
                         preferred_element_type=jnp.float32) + b_ref[...]


def _combine(partials, b):
    return pl.pallas_call(
        _combine_body,
        grid=(N // _BN,),
        in_specs=[
            pl.BlockSpec((NC, _BN, D), lambda i: (0, i, 0)),
            pl.BlockSpec((D, D), lambda i: (0, 0)),
            pl.BlockSpec((1, D), lambda i: (0, 0)),
        ],
        out_specs=pl.BlockSpec((_BN, D), lambda i: (i, 0)),
        out_shape=jax.ShapeDtypeStruct((N, D), jnp.float32),
    )(partials, jnp.asarray(_PMAT), b.reshape(1, D))


def kernel(input, edge_index, edge_weight, W, b):
    support_bf = _matmul(input, W)
    partials = _sc_aggregate(support_bf, edge_index[0], edge_index[1],
                             edge_weight)
    return _combine(partials, b)


# speedup vs baseline: 1.0571x; 1.0571x over previous
import functools

import jax
import jax.numpy as jnp
from jax import lax
from jax.experimental import pallas as pl
from jax.experimental.pallas import tpu as pltpu
from jax.experimental.pallas import tpu_sc as plsc

N = 10000
E = 320000
D = 128

NC = 2
NS = 16
NW = NC * NS
EPT = E // NW
K = 40
NB = EPT // K
RPT = 624
REM = N - NS * RPT
IDXB = 14


_BN = 1000


def _matmul_body(x_ref, w_ref, o_ref):
    o_ref[...] = jnp.dot(x_ref[...], w_ref[...],
                         preferred_element_type=jnp.float32)


def _matmul(x, W):
    return pl.pallas_call(
        _matmul_body,
        grid=(N // _BN,),
        in_specs=[
            pl.BlockSpec((_BN, D), lambda i: (i, 0)),
            pl.BlockSpec((D, D), lambda i: (0, 0)),
        ],
        out_specs=pl.BlockSpec((_BN, D), lambda i: (i, 0)),
        out_shape=jax.ShapeDtypeStruct((N, D), jnp.float32),
    )(x, W)


def _scale_rows(rows_v, w_all, bi):
    for g in range((K + 15) // 16):
        p = min(16, K - g * 16)
        lane0 = 16 - p
        wv = w_all[pl.ds(bi * K + g * 16 - lane0, 16)]
        for t in range(p):
            wj = wv[lane0 + t]
            j = g * 16 + t
            for cc in range(D // 16):
                sl = pl.ds(cc * 16, 16)
                rows_v[j, sl] = rows_v[j, sl] * wj


def _sc_body(support_hbm, sd_hbm, w_hbm, zeros_hbm, out_hbm,
             acc, sd_all, w_all, rb0, rb1, rb2, rb3,
             sblk, dblk, psem, g0, g1, g2, g3, t0, t1, t2, t3):
    c = lax.axis_index("c")
    s = lax.axis_index("s")
    wid = c * NS + s
    rbs = (rb0, rb1, rb2, rb3)
    gs = (g0, g1, g2, g3)
    ts = (t0, t1, t2, t3)

    d1 = pltpu.async_copy(sd_hbm.at[wid], sd_all, psem)
    d2 = pltpu.async_copy(w_hbm.at[wid], w_all, psem)
    d3 = pltpu.async_copy(zeros_hbm, acc.at[pl.ds(s * RPT, RPT)], psem)
    pltpu.async_copy(zeros_hbm.at[pl.ds(0, K)], rb3, t3)

    @pl.when(s == 0)
    def _():
        pltpu.sync_copy(zeros_hbm.at[pl.ds(0, REM)],
                        acc.at[pl.ds(NS * RPT, REM)])

    def unpack_src(bi, u):
        for off in (0, 16, K - 16):
            v = sd_all[bi, pl.ds(off, 16)]
            sblk[u, pl.ds(off, 16)] = v & ((1 << IDXB) - 1)

    def unpack_dst(bi, u):
        for off in (0, 16, K - 16):
            v = sd_all[bi, pl.ds(off, 16)]
            dblk[u, pl.ds(off, 16)] = v >> IDXB

    def gather(bi, u):
        pltpu.async_copy(support_hbm.at[sblk.at[u]], rbs[u], gs[u])

    def drain_gather(u):
        pltpu.make_async_copy(support_hbm.at[pl.ds(0, K)], rbs[u],
                              gs[u]).wait()

    def scatter(bi, u):
        pltpu.async_copy(rbs[u], acc.at[dblk.at[u]], ts[u], add=True)

    def drain_scatter(u):
        pltpu.make_async_copy(zeros_hbm.at[pl.ds(0, K)], rbs[u],
                              ts[u]).wait()

    d1.wait()
    d2.wait()
    d3.wait()
    for u in range(3):
        unpack_src(u, u)
        gather(u, u)
    plsc.subcore_barrier()

    def block_body(m, u):
        un = (u + 3) % 4
        drain_gather(u)
        _scale_rows(rbs[u], w_all, m)
        drain_scatter(un)

        @pl.when(m + 3 < NB)
        def _():
            unpack_src(m + 3, un)
            gather(m + 3, un)

        unpack_dst(m, u)
        scatter(m, u)

    block_body(0, 0)
    block_body(1, 1)

    def quad(t, _):
        for i in range(4):
            block_body(4 * t + i + 2, (i + 2) % 4)
        return 0

    lax.fori_loop(0, (NB - 2) // 4, quad, 0)
    drain_scatter((NB - 1) % 4)
    plsc.subcore_barrier()

    pltpu.sync_copy(acc.at[pl.ds(s * RPT, RPT)],
                    out_hbm.at[c, pl.ds(s * RPT, RPT)])

    @pl.when(s == 0)
    def _():
        pltpu.sync_copy(acc.at[pl.ds(NS * RPT, REM)],
                        out_hbm.at[c, pl.ds(NS * RPT, REM)])


def _sc_aggregate(support, src, dst, w):
    mesh = plsc.VectorSubcoreMesh(core_axis_name="c", subcore_axis_name="s",
                                  num_cores=NC, num_subcores=NS)
    sd = src | (dst << IDXB)
    call = pl.kernel(
        _sc_body,
        out_type=jax.ShapeDtypeStruct((NC, N, D), jnp.float32),
        mesh=mesh,
        compiler_params=pltpu.CompilerParams(use_tc_tiling_on_sc=False),
        scratch_types=[
            pltpu.VMEM_SHARED((N, D), jnp.float32),
            pltpu.VMEM((NB, K), jnp.int32),
            pltpu.VMEM((EPT,), jnp.float32),
            pltpu.VMEM((K, D), jnp.float32),
            pltpu.VMEM((K, D), jnp.float32),
            pltpu.VMEM((K, D), jnp.float32),
            pltpu.VMEM((K, D), jnp.float32),
            pltpu.VMEM((4, K), jnp.int32),
            pltpu.VMEM((4, K), jnp.int32),
            pltpu.SemaphoreType.DMA,
            pltpu.SemaphoreType.DMA,
            pltpu.SemaphoreType.DMA,
            pltpu.SemaphoreType.DMA,
            pltpu.SemaphoreType.DMA,
            pltpu.SemaphoreType.DMA,
            pltpu.SemaphoreType.DMA,
            pltpu.SemaphoreType.DMA,
            pltpu.SemaphoreType.DMA,
        ],
    )
    return call(support,
                sd.reshape(NW, NB, K),
                w.reshape(NW, EPT),
                jnp.zeros((RPT, D), jnp.float32))


def _combine_body(p_ref, b_ref, o_ref):
    o_ref[...] = p_ref[0] + p_ref[1] + b_ref[...]


def _combine(partials, b):
    return pl.pallas_call(
        _combine_body,
        grid=(N // _BN,),
        in_specs=[
            pl.BlockSpec((NC, _BN, D), lambda i: (0, i, 0)),
            pl.BlockSpec((1, D), lambda i: (0, 0)),
        ],
        out_specs=pl.BlockSpec((_BN, D), lambda i: (i, 0)),
        out_shape=jax.ShapeDtypeStruct((N, D), jnp.float32),
    )(partials, b.reshape(1, D))


def kernel(input, edge_index, edge_weight, W, b):
    support = _matmul(input, W)
    partials = _sc_aggregate(support, edge_index[0], edge_index[1],
                             edge_weight)
    return _combine(partials, b)


# speedup vs baseline: 1.1062x; 1.0464x over previous
import functools

import jax
import jax.numpy as jnp
from jax import lax
from jax.experimental import pallas as pl
from jax.experimental.pallas import tpu as pltpu
from jax.experimental.pallas import tpu_sc as plsc

N = 10000
E = 320000
D = 128

NC = 2
NS = 16
NW = NC * NS
EPT = E // NW
K = 40
NB = EPT // K
RPT = 624
REM = N - NS * RPT
IDXB = 14

_BN = 1000


def _scale_rows(rows_v, w_all, bi):
    for g in range((K + 15) // 16):
        p = min(16, K - g * 16)
        lane0 = 16 - p
        wv = w_all[pl.ds(bi * K + g * 16 - lane0, 16)]
        for t in range(p):
            wj = wv[lane0 + t]
            j = g * 16 + t
            for cc in range(D // 16):
                sl = pl.ds(cc * 16, 16)
                rows_v[j, sl] = rows_v[j, sl] * wj


def _sc_body(support_hbm, sd_hbm, w_hbm, zeros_hbm, out_hbm,
             acc, sd_all, w_all, rb0, rb1, rb2, rb3,
             sblk, dblk, psem, g0, g1, g2, g3, t0, t1, t2, t3):
    c = lax.axis_index("c")
    s = lax.axis_index("s")
    wid = c * NS + s
    rbs = (rb0, rb1, rb2, rb3)
    gs = (g0, g1, g2, g3)
    ts = (t0, t1, t2, t3)

    d1 = pltpu.async_copy(sd_hbm.at[wid], sd_all, psem)
    d2 = pltpu.async_copy(w_hbm.at[wid], w_all, psem)
    d3 = pltpu.async_copy(zeros_hbm, acc.at[pl.ds(s * RPT, RPT)], psem)
    pltpu.async_copy(zeros_hbm.at[pl.ds(0, K)], rb3, t3)

    @pl.when(s == 0)
    def _():
        pltpu.sync_copy(zeros_hbm.at[pl.ds(0, REM)],
                        acc.at[pl.ds(NS * RPT, REM)])

    def unpack_src(bi, u):
        for off in (0, 16, K - 16):
            v = sd_all[bi, pl.ds(off, 16)]
            sblk[u, pl.ds(off, 16)] = v & ((1 << IDXB) - 1)

    def unpack_dst(bi, u):
        for off in (0, 16, K - 16):
            v = sd_all[bi, pl.ds(off, 16)]
            dblk[u, pl.ds(off, 16)] = v >> IDXB

    def gather(bi, u):
        pltpu.async_copy(support_hbm.at[sblk.at[u]], rbs[u], gs[u])

    def drain_gather(u):
        pltpu.make_async_copy(support_hbm.at[pl.ds(0, K)], rbs[u],
                              gs[u]).wait()

    def scatter(bi, u):
        pltpu.async_copy(rbs[u], acc.at[dblk.at[u]], ts[u], add=True)

    def drain_scatter(u):
        pltpu.make_async_copy(zeros_hbm.at[pl.ds(0, K)], rbs[u],
                              ts[u]).wait()

    d1.wait()
    d2.wait()
    d3.wait()
    for u in range(3):
        unpack_src(u, u)
        gather(u, u)
    plsc.subcore_barrier()

    def block_body(m, u):
        un = (u + 3) % 4
        drain_gather(u)
        _scale_rows(rbs[u], w_all, m)
        drain_scatter(un)

        @pl.when(m + 3 < NB)
        def _():
            unpack_src(m + 3, un)
            gather(m + 3, un)

        unpack_dst(m, u)
        scatter(m, u)

    block_body(0, 0)
    block_body(1, 1)

    def quad(t, _):
        for i in range(4):
            block_body(4 * t + i + 2, (i + 2) % 4)
        return 0

    lax.fori_loop(0, (NB - 2) // 4, quad, 0)
    drain_scatter((NB - 1) % 4)
    plsc.subcore_barrier()

    pltpu.sync_copy(acc.at[pl.ds(s * RPT, RPT)],
                    out_hbm.at[c, pl.ds(s * RPT, RPT)])

    @pl.when(s == 0)
    def _():
        pltpu.sync_copy(acc.at[pl.ds(NS * RPT, REM)],
                        out_hbm.at[c, pl.ds(NS * RPT, REM)])


def _sc_aggregate(support, src, dst, w):
    mesh = plsc.VectorSubcoreMesh(core_axis_name="c", subcore_axis_name="s",
                                  num_cores=NC, num_subcores=NS)
    sd = src | (dst << IDXB)
    call = pl.kernel(
        _sc_body,
        out_type=jax.ShapeDtypeStruct((NC, N, D), jnp.float32),
        mesh=mesh,
        compiler_params=pltpu.CompilerParams(use_tc_tiling_on_sc=False),
        scratch_types=[
            pltpu.VMEM_SHARED((N, D), jnp.float32),
            pltpu.VMEM((NB, K), jnp.int32),
            pltpu.VMEM((EPT,), jnp.float32),
            pltpu.VMEM((K, D), jnp.float32),
            pltpu.VMEM((K, D), jnp.float32),
            pltpu.VMEM((K, D), jnp.float32),
            pltpu.VMEM((K, D), jnp.float32),
            pltpu.VMEM((4, K), jnp.int32),
            pltpu.VMEM((4, K), jnp.int32),
            pltpu.SemaphoreType.DMA,
            pltpu.SemaphoreType.DMA,
            pltpu.SemaphoreType.DMA,
            pltpu.SemaphoreType.DMA,
            pltpu.SemaphoreType.DMA,
            pltpu.SemaphoreType.DMA,
            pltpu.SemaphoreType.DMA,
            pltpu.SemaphoreType.DMA,
            pltpu.SemaphoreType.DMA,
        ],
    )
    return call(support,
                sd.reshape(NW, NB, K),
                w.reshape(NW, EPT),
                jnp.zeros((RPT, D), jnp.float32))


def _combine_body(p_ref, w_ref, b_ref, o_ref):
    o_ref[...] = jnp.dot(p_ref[0] + p_ref[1], w_ref[...],
                         preferred_element_type=jnp.float32) + b_ref[...]


def _combine(partials, W, b):
    return pl.pallas_call(
        _combine_body,
        grid=(N // _BN,),
        in_specs=[
            pl.BlockSpec((NC, _BN, D), lambda i: (0, i, 0)),
            pl.BlockSpec((D, D), lambda i: (0, 0)),
            pl.BlockSpec((1, D), lambda i: (0, 0)),
        ],
        out_specs=pl.BlockSpec((_BN, D), lambda i: (i, 0)),
        out_shape=jax.ShapeDtypeStruct((N, D), jnp.float32),
    )(partials, W, b.reshape(1, D))


def kernel(input, edge_index, edge_weight, W, b):
    partials = _sc_aggregate(input, edge_index[0], edge_index[1],
                             edge_weight)
    return _combine(partials, W, b)


# speedup vs baseline: 1.1090x; 1.0025x over previous
import functools

import jax
import jax.numpy as jnp
from jax import lax
from jax.experimental import pallas as pl
from jax.experimental.pallas import tpu as pltpu
from jax.experimental.pallas import tpu_sc as plsc

N = 10000
E = 320000
D = 128

NC = 2
NS = 16
NW = NC * NS
EPT = E // NW
K = 40
NB = EPT // K
RPT = 624
REM = N - NS * RPT
IDXB = 14

_BN = 1000


def _scale_rows(rows_v, w_all, bi):
    for g in range((K + 15) // 16):
        p = min(16, K - g * 16)
        lane0 = 16 - p
        wv = w_all[pl.ds(bi * K + g * 16 - lane0, 16)]
        for t in range(p):
            wj = wv[lane0 + t]
            j = g * 16 + t
            for cc in range(D // 16):
                sl = pl.ds(cc * 16, 16)
                rows_v[j, sl] = rows_v[j, sl] * wj


def _sc_body(support_hbm, sd_hbm, w_hbm, zeros_hbm, out_hbm,
             acc, sd_all, w_all, rb0, rb1, rb2, rb3,
             sblk, dblk, psem, g0, g1, g2, g3, t0, t1, t2, t3):
    c = lax.axis_index("c")
    s = lax.axis_index("s")
    wid = c * NS + s
    rbs = (rb0, rb1, rb2, rb3)
    gs = (g0, g1, g2, g3)
    ts = (t0, t1, t2, t3)

    d1 = pltpu.async_copy(sd_hbm.at[pl.ds(wid * EPT, EPT)], sd_all, psem)
    d2 = pltpu.async_copy(w_hbm.at[pl.ds(wid * EPT, EPT)], w_all, psem)
    d3 = pltpu.async_copy(zeros_hbm, acc.at[pl.ds(s * RPT, RPT)], psem)
    pltpu.async_copy(zeros_hbm.at[pl.ds(0, K)], rb3, t3)

    @pl.when(s == 0)
    def _():
        pltpu.sync_copy(zeros_hbm.at[pl.ds(0, REM)],
                        acc.at[pl.ds(NS * RPT, REM)])

    def unpack_src(bi, u):
        for off in (0, 16, K - 16):
            v = sd_all[pl.ds(bi * K + off, 16)]
            sblk[u, pl.ds(off, 16)] = v & ((1 << IDXB) - 1)

    def unpack_dst(bi, u):
        for off in (0, 16, K - 16):
            v = sd_all[pl.ds(bi * K + off, 16)]
            dblk[u, pl.ds(off, 16)] = v >> IDXB

    def gather(bi, u):
        pltpu.async_copy(support_hbm.at[sblk.at[u]], rbs[u], gs[u])

    def drain_gather(u):
        pltpu.make_async_copy(support_hbm.at[pl.ds(0, K)], rbs[u],
                              gs[u]).wait()

    def scatter(bi, u):
        pltpu.async_copy(rbs[u], acc.at[dblk.at[u]], ts[u], add=True)

    def drain_scatter(u):
        pltpu.make_async_copy(zeros_hbm.at[pl.ds(0, K)], rbs[u],
                              ts[u]).wait()

    d1.wait()
    d2.wait()
    d3.wait()
    for u in range(3):
        unpack_src(u, u)
        gather(u, u)
    plsc.subcore_barrier()

    def block_body(m, u):
        un = (u + 3) % 4
        drain_gather(u)
        _scale_rows(rbs[u], w_all, m)
        drain_scatter(un)

        @pl.when(m + 3 < NB)
        def _():
            unpack_src(m + 3, un)
            gather(m + 3, un)

        unpack_dst(m, u)
        scatter(m, u)

    block_body(0, 0)
    block_body(1, 1)

    def quad(t, _):
        for i in range(4):
            block_body(4 * t + i + 2, (i + 2) % 4)
        return 0

    lax.fori_loop(0, (NB - 2) // 4, quad, 0)
    drain_scatter((NB - 1) % 4)
    plsc.subcore_barrier()

    pltpu.sync_copy(acc.at[pl.ds(s * RPT, RPT)],
                    out_hbm.at[c, pl.ds(s * RPT, RPT)])

    @pl.when(s == 0)
    def _():
        pltpu.sync_copy(acc.at[pl.ds(NS * RPT, REM)],
                        out_hbm.at[c, pl.ds(NS * RPT, REM)])


def _sc_aggregate(support, src, dst, w):
    mesh = plsc.VectorSubcoreMesh(core_axis_name="c", subcore_axis_name="s",
                                  num_cores=NC, num_subcores=NS)
    sd = src | (dst << IDXB)
    call = pl.kernel(
        _sc_body,
        out_type=jax.ShapeDtypeStruct((NC, N, D), jnp.float32),
        mesh=mesh,
        scratch_types=[
            pltpu.VMEM_SHARED((N, D), jnp.float32),
            pltpu.VMEM((EPT,), jnp.int32),
            pltpu.VMEM((EPT,), jnp.float32),
            pltpu.VMEM((K, D), jnp.float32),
            pltpu.VMEM((K, D), jnp.float32),
            pltpu.VMEM((K, D), jnp.float32),
            pltpu.VMEM((K, D), jnp.float32),
            pltpu.VMEM((4, K), jnp.int32),
            pltpu.VMEM((4, K), jnp.int32),
            pltpu.SemaphoreType.DMA,
            pltpu.SemaphoreType.DMA,
            pltpu.SemaphoreType.DMA,
            pltpu.SemaphoreType.DMA,
            pltpu.SemaphoreType.DMA,
            pltpu.SemaphoreType.DMA,
            pltpu.SemaphoreType.DMA,
            pltpu.SemaphoreType.DMA,
            pltpu.SemaphoreType.DMA,
        ],
    )
    return call(support, sd, w, jnp.zeros((RPT, D), jnp.float32))


def _combine_body(p_ref, w_ref, b_ref, o_ref):
    o_ref[...] = jnp.dot(p_ref[0] + p_ref[1], w_ref[...],
                         preferred_element_type=jnp.float32) + b_ref[...]


def _combine(partials, W, b):
    return pl.pallas_call(
        _combine_body,
        grid=(N // _BN,),
        in_specs=[
            pl.BlockSpec((NC, _BN, D), lambda i: (0, i, 0)),
            pl.BlockSpec((D, D), lambda i: (0, 0)),
            pl.BlockSpec((1, D), lambda i: (0, 0)),
        ],
        out_specs=pl.BlockSpec((_BN, D), lambda i: (i, 0)),
        out_shape=jax.ShapeDtypeStruct((N, D), jnp.float32),
    )(partials, W, b.reshape(1, D))


def kernel(input, edge_index, edge_weight, W, b):
    partials = _sc_aggregate(input, edge_index[0], edge_index[1],
                             edge_weight)
    return _combine(partials, W, b)
